# baseline (speedup 1.0000x reference)
import dataclasses
import functools

import jax
import jax.numpy as jnp
from jax import lax
from jax.experimental import pallas as pl
from jax.experimental.pallas import tpu as pltpu
from jax.experimental.pallas import tpu_sc as plsc

N = 10000
D = 128
DE = 144
E = 320000
BN = 1000
CH = 128
NSUB = 16
NCHP = 2512
EPAD = NCHP * CH
NP = 10112
ROWS_PER_SUB = NP // NSUB
PAD_ROW = NP - 8


def _prep_body(x_ref, w_ref, att_ref, xme_l_ref, xme_u_ref, a_l_ref, a_u_ref):
    x = x_ref[...]
    ones = jnp.ones((x.shape[0], 1), jnp.float32)
    zeros = jnp.zeros((x.shape[0], DE - D - 1), jnp.float32)
    xm_l = jnp.dot(x, w_ref[0], preferred_element_type=jnp.float32)
    xm_u = jnp.dot(x, w_ref[1], preferred_element_type=jnp.float32)
    xme_l_ref[...] = jnp.concatenate([xm_l, ones, zeros], axis=1)
    xme_u_ref[...] = jnp.concatenate([xm_u, ones, zeros], axis=1)
    a_l_ref[...] = jnp.dot(xm_l, att_ref[0], preferred_element_type=jnp.float32)
    a_u_ref[...] = jnp.dot(xm_u, att_ref[1], preferred_element_type=jnp.float32)


def _prep(x, W2, att2):
    nb = N // BN
    return pl.pallas_call(
        _prep_body,
        grid=(nb,),
        in_specs=[
            pl.BlockSpec((BN, D), lambda b: (b, 0)),
            pl.BlockSpec((2, D, D), lambda b: (0, 0, 0)),
            pl.BlockSpec((2, D, 2), lambda b: (0, 0, 0)),
        ],
        out_specs=[
            pl.BlockSpec((BN, DE), lambda b: (b, 0)),
            pl.BlockSpec((BN, DE), lambda b: (b, 0)),
            pl.BlockSpec((BN, 2), lambda b: (b, 0)),
            pl.BlockSpec((BN, 2), lambda b: (b, 0)),
        ],
        out_shape=[
            jax.ShapeDtypeStruct((N, DE), jnp.float32),
            jax.ShapeDtypeStruct((N, DE), jnp.float32),
            jax.ShapeDtypeStruct((N, 2), jnp.float32),
            jax.ShapeDtypeStruct((N, 2), jnp.float32),
        ],
    )(x, W2, att2)


_GDN = lax.GatherDimensionNumbers(offset_dims=(), collapsed_slice_dims=(0,),
                                  start_index_map=(0,))


def _bcast16(vec, e):
    idx = jnp.full((16, 1), e, jnp.int32)
    return lax.gather(vec, idx, _GDN, (1,),
                      mode=lax.GatherScatterMode.PROMISE_IN_BOUNDS)

def _sc_conv(s, xme_hbm, edata_hbm, a_hbm, zeros_hbm,
             out_hbm, acc_s, a_src_v, a_tgt_v, ebuf0, rows_v, gsems, wsems):
    pltpu.sync_copy(a_hbm.at[0], a_src_v)
    pltpu.sync_copy(a_hbm.at[1], a_tgt_v)
    row0 = s * ROWS_PER_SUB
    pltpu.sync_copy(zeros_hbm.at[pl.ds(row0, ROWS_PER_SUB)],
                    acc_s.at[pl.ds(row0, ROWS_PER_SUB)])
    plsc.subcore_barrier()

    HH = CH // 2

    @pl.loop(s, NCHP, step=NSUB)
    def _chunk(k):
        pltpu.sync_copy(edata_hbm.at[k], ebuf0)
        gathers = []
        for h in range(2):
            g = pltpu.make_async_copy(
                xme_hbm.at[ebuf0.at[2 + h]],
                rows_v.at[pl.ds(h * HH, HH)], gsems[h])
            g.start()
            gathers.append(g)
        scatters = []
        for h in range(2):
            tgt_r = ebuf0.at[h]
            src_r = ebuf0.at[2 + h]
            vals_r = ebuf0.at[4 + h]
            gathers[h].wait()

            for g in range(HH // 16):
                sl = pl.ds(g * 16, 16)
                a_s = plsc.load_gather(a_src_v, [src_r[sl]])
                a_t = plsc.load_gather(a_tgt_v, [tgt_r[sl]])
                z = a_s + a_t
                att = jnp.where(z > 0, z, jnp.exp(z) - 1.0)
                ev = jnp.exp(att * plsc.bitcast(vals_r[sl], jnp.float32))
                for e in range(16):
                    bc = _bcast16(ev, e)
                    row = rows_v.at[h * HH + g * 16 + e]
                    for j in range(DE // 16):
                        jl = pl.ds(j * 16, 16)
                        row[jl] = row[jl] * bc

            w = pltpu.make_async_copy(rows_v.at[pl.ds(h * HH, HH)],
                                      acc_s.at[tgt_r], wsems[h])
            w.start(add=True)
            scatters.append(w)
        for w in scatters:
            w.wait()

    plsc.subcore_barrier()
    pltpu.sync_copy(acc_s.at[pl.ds(row0, ROWS_PER_SUB)],
                    out_hbm.at[pl.ds(row0, ROWS_PER_SUB)])


def _sc_sparse(xme_l, xme_u, a_l, a_u, edata_l, edata_u, zeros):
    mesh = plsc.VectorSubcoreMesh(core_axis_name="c", subcore_axis_name="s")
    cp = pltpu.CompilerParams(needs_layout_passes=False,
                              use_tc_tiling_on_sc=False)

    @functools.partial(
        pl.kernel,
        compiler_params=cp,
        out_type=[jax.ShapeDtypeStruct((NP, DE), jnp.float32),
                  jax.ShapeDtypeStruct((NP, DE), jnp.float32)],
        mesh=mesh,
        scratch_types=[
            pltpu.VMEM_SHARED((NP, DE), jnp.float32),
            pltpu.VMEM((N,), jnp.float32),
            pltpu.VMEM((N,), jnp.float32),
            pltpu.VMEM((6, CH // 2), jnp.int32),
            pltpu.VMEM((CH, DE), jnp.float32),
            pltpu.SemaphoreType.DMA((2,)),
            pltpu.SemaphoreType.DMA((2,)),
        ],
    )
    def body(xme_l_h, xme_u_h, a_l_h, a_u_h, edata_l_h, edata_u_h, zeros_h,
             out_l_h, out_u_h, acc_s, a_src_v, a_tgt_v, ebuf0, rows_v,
             gsem_arr, wsem_arr):
        c = lax.axis_index("c")
        s = lax.axis_index("s")

        @pl.when(c == 0)
        def _():
            _sc_conv(s, xme_l_h, edata_l_h, a_l_h, zeros_h,
                     out_l_h, acc_s, a_src_v, a_tgt_v, ebuf0, rows_v,
                     [gsem_arr.at[0], gsem_arr.at[1]],
                     [wsem_arr.at[0], wsem_arr.at[1]])

        @pl.when(c == 1)
        def _():
            _sc_conv(s, xme_u_h, edata_u_h, a_u_h, zeros_h,
                     out_u_h, acc_s, a_src_v, a_tgt_v, ebuf0, rows_v,
                     [gsem_arr.at[0], gsem_arr.at[1]],
                     [wsem_arr.at[0], wsem_arr.at[1]])

    return body(xme_l, xme_u, a_l, a_u, edata_l, edata_u, zeros)


def _pack_edges(indices, values):
    npad = EPAD - E
    tgt = jnp.concatenate([indices[0], jnp.full((npad,), PAD_ROW, jnp.int32)])
    src = jnp.concatenate([indices[1], jnp.zeros((npad,), jnp.int32)])
    vbits = lax.bitcast_convert_type(
        jnp.concatenate([values, jnp.zeros((npad,), jnp.float32)]), jnp.int32)
    hh = CH // 2
    return jnp.concatenate([tgt.reshape(-1, 2, hh), src.reshape(-1, 2, hh),
                            vbits.reshape(-1, 2, hh)], axis=1)


def _combine_body(accl_ref, accu_ref, x_ref, wlt_ref, out_ref):
    accl = accl_ref[...]
    accu = accu_ref[...]
    sl = accl[:, D:D + 1]
    su = accu[:, D:D + 1]
    invl = jnp.where(sl > 0, 1.0 / sl, 0.0)
    invu = jnp.where(su > 0, 1.0 / su, 0.0)
    skip = jnp.dot(x_ref[...], wlt_ref[...], preferred_element_type=jnp.float32)
    out_ref[...] = jax.nn.relu(
        accl[:, :D] * invl + accu[:, :D] * invu + skip * (1.0 + 1e-6))


def _combine(acc_l, acc_u, x, W_lin_T):
    nb = N // BN
    return pl.pallas_call(
        _combine_body,
        grid=(nb,),
        in_specs=[
            pl.BlockSpec((BN, DE), lambda b: (b, 0)),
            pl.BlockSpec((BN, DE), lambda b: (b, 0)),
            pl.BlockSpec((BN, D), lambda b: (b, 0)),
            pl.BlockSpec((D, D), lambda b: (0, 0)),
        ],
        out_specs=pl.BlockSpec((BN, D), lambda b: (b, 0)),
        out_shape=jax.ShapeDtypeStruct((N, D), jnp.float32),
    )(acc_l, acc_u, x, W_lin_T)


def kernel(x, lower_indices, lower_values, upper_indices, upper_values,
           W_lower, att_lower, W_upper, att_upper, W_lin):
    W2 = jnp.stack([W_lower, W_upper])
    att2 = jnp.stack([att_lower.reshape(2, D).T, att_upper.reshape(2, D).T])
    xme_l, xme_u, a_l, a_u = _prep(x, W2, att2)
    a_l_t = jnp.transpose(a_l)
    a_u_t = jnp.transpose(a_u)
    zeros = jnp.zeros((NP, DE), jnp.float32)
    edata_l = _pack_edges(lower_indices, lower_values)
    edata_u = _pack_edges(upper_indices, upper_values)
    acc_l, acc_u = _sc_sparse(xme_l, xme_u, a_l_t, a_u_t,
                              edata_l, edata_u, zeros)
    return _combine(acc_l, acc_u, x, W_lin.T)


# speedup vs baseline: 1.0005x; 1.0005x over previous
import functools

import jax
import jax.numpy as jnp
from jax import lax
from jax.experimental import pallas as pl
from jax.experimental.pallas import tpu as pltpu
from jax.experimental.pallas import tpu_sc as plsc

N = 10000
D = 128
DE = 144
E = 320000
BN = 1000
CH = 128
NSUB = 16
NCHP = 2512
EPAD = NCHP * CH
NP = 10112
ROWS_PER_SUB = NP // NSUB
PAD_ROW = NP - 8


def _prep_body(x_ref, w_ref, att_ref, xme_l_ref, xme_u_ref, a_l_ref, a_u_ref):
    x = x_ref[...]
    ones = jnp.ones((x.shape[0], 1), jnp.float32)
    zeros = jnp.zeros((x.shape[0], DE - D - 1), jnp.float32)
    xm_l = jnp.dot(x, w_ref[0], preferred_element_type=jnp.float32)
    xm_u = jnp.dot(x, w_ref[1], preferred_element_type=jnp.float32)
    xme_l_ref[...] = jnp.concatenate([xm_l, ones, zeros], axis=1)
    xme_u_ref[...] = jnp.concatenate([xm_u, ones, zeros], axis=1)
    a_l_ref[...] = jnp.dot(xm_l, att_ref[0], preferred_element_type=jnp.float32)
    a_u_ref[...] = jnp.dot(xm_u, att_ref[1], preferred_element_type=jnp.float32)


def _prep(x, W2, att2):
    nb = N // BN
    return pl.pallas_call(
        _prep_body,
        grid=(nb,),
        in_specs=[
            pl.BlockSpec((BN, D), lambda b: (b, 0)),
            pl.BlockSpec((2, D, D), lambda b: (0, 0, 0)),
            pl.BlockSpec((2, D, 2), lambda b: (0, 0, 0)),
        ],
        out_specs=[
            pl.BlockSpec((BN, DE), lambda b: (b, 0)),
            pl.BlockSpec((BN, DE), lambda b: (b, 0)),
            pl.BlockSpec((BN, 2), lambda b: (b, 0)),
            pl.BlockSpec((BN, 2), lambda b: (b, 0)),
        ],
        out_shape=[
            jax.ShapeDtypeStruct((N, DE), jnp.float32),
            jax.ShapeDtypeStruct((N, DE), jnp.float32),
            jax.ShapeDtypeStruct((N, 2), jnp.float32),
            jax.ShapeDtypeStruct((N, 2), jnp.float32),
        ],
    )(x, W2, att2)


_GDN = lax.GatherDimensionNumbers(offset_dims=(), collapsed_slice_dims=(0,),
                                  start_index_map=(0,))


def _bcast16(vec, e):
    idx = jnp.full((16, 1), e, jnp.int32)
    return lax.gather(vec, idx, _GDN, (1,),
                      mode=lax.GatherScatterMode.PROMISE_IN_BOUNDS)

def _sc_conv(s, xme_hbm, edata_hbm, a_hbm, zeros_hbm,
             out_hbm, acc_s, a_src_v, a_tgt_v, ebuf0, rows_v, gsems, wsems):
    pltpu.sync_copy(a_hbm.at[0], a_src_v)
    pltpu.sync_copy(a_hbm.at[1], a_tgt_v)
    row0 = s * ROWS_PER_SUB
    pltpu.sync_copy(zeros_hbm.at[pl.ds(row0, ROWS_PER_SUB)],
                    acc_s.at[pl.ds(row0, ROWS_PER_SUB)])
    plsc.subcore_barrier()

    HH = CH // 2

    @pl.loop(s, NCHP, step=NSUB)
    def _chunk(k):
        pltpu.sync_copy(edata_hbm.at[k], ebuf0)
        gathers = []
        for h in range(2):
            g = pltpu.make_async_copy(
                xme_hbm.at[ebuf0.at[2 + h]],
                rows_v.at[pl.ds(h * HH, HH)], gsems[h])
            g.start()
            gathers.append(g)
        scatters = []
        for h in range(2):
            tgt_r = ebuf0.at[h]
            src_r = ebuf0.at[2 + h]
            vals_r = ebuf0.at[4 + h]
            gathers[h].wait()

            @pl.loop(0, HH // 16)
            def _group(g):
                sl = pl.ds(g * 16, 16)
                a_s = plsc.load_gather(a_src_v, [src_r[sl]])
                a_t = plsc.load_gather(a_tgt_v, [tgt_r[sl]])
                z = a_s + a_t
                att = jnp.where(z > 0, z, jnp.exp(z) - 1.0)
                ev = jnp.exp(att * plsc.bitcast(vals_r[sl], jnp.float32))
                for e in range(16):
                    bc = _bcast16(ev, e)
                    row = rows_v.at[h * HH + g * 16 + e]
                    for j in range(DE // 16):
                        jl = pl.ds(j * 16, 16)
                        row[jl] = row[jl] * bc

            w = pltpu.make_async_copy(rows_v.at[pl.ds(h * HH, HH)],
                                      acc_s.at[tgt_r], wsems[h])
            w.start(add=True)
            scatters.append(w)
        for w in scatters:
            w.wait()

    plsc.subcore_barrier()
    pltpu.sync_copy(acc_s.at[pl.ds(row0, ROWS_PER_SUB)],
                    out_hbm.at[pl.ds(row0, ROWS_PER_SUB)])


def _sc_sparse(xme_l, xme_u, a_l, a_u, edata_l, edata_u, zeros):
    mesh = plsc.VectorSubcoreMesh(core_axis_name="c", subcore_axis_name="s")
    cp = pltpu.CompilerParams(needs_layout_passes=False,
                              use_tc_tiling_on_sc=False)

    @functools.partial(
        pl.kernel,
        compiler_params=cp,
        out_type=[jax.ShapeDtypeStruct((NP, DE), jnp.float32),
                  jax.ShapeDtypeStruct((NP, DE), jnp.float32)],
        mesh=mesh,
        scratch_types=[
            pltpu.VMEM_SHARED((NP, DE), jnp.float32),
            pltpu.VMEM((N,), jnp.float32),
            pltpu.VMEM((N,), jnp.float32),
            pltpu.VMEM((6, CH // 2), jnp.int32),
            pltpu.VMEM((CH, DE), jnp.float32),
            pltpu.SemaphoreType.DMA((2,)),
            pltpu.SemaphoreType.DMA((2,)),
        ],
    )
    def body(xme_l_h, xme_u_h, a_l_h, a_u_h, edata_l_h, edata_u_h, zeros_h,
             out_l_h, out_u_h, acc_s, a_src_v, a_tgt_v, ebuf0, rows_v,
             gsem_arr, wsem_arr):
        c = lax.axis_index("c")
        s = lax.axis_index("s")

        @pl.when(c == 0)
        def _():
            _sc_conv(s, xme_l_h, edata_l_h, a_l_h, zeros_h,
                     out_l_h, acc_s, a_src_v, a_tgt_v, ebuf0, rows_v,
                     [gsem_arr.at[0], gsem_arr.at[1]],
                     [wsem_arr.at[0], wsem_arr.at[1]])

        @pl.when(c == 1)
        def _():
            _sc_conv(s, xme_u_h, edata_u_h, a_u_h, zeros_h,
                     out_u_h, acc_s, a_src_v, a_tgt_v, ebuf0, rows_v,
                     [gsem_arr.at[0], gsem_arr.at[1]],
                     [wsem_arr.at[0], wsem_arr.at[1]])

    return body(xme_l, xme_u, a_l, a_u, edata_l, edata_u, zeros)


def _pack_edges(indices, values):
    npad = EPAD - E
    tgt = jnp.concatenate([indices[0], jnp.full((npad,), PAD_ROW, jnp.int32)])
    src = jnp.concatenate([indices[1], jnp.zeros((npad,), jnp.int32)])
    vbits = lax.bitcast_convert_type(
        jnp.concatenate([values, jnp.zeros((npad,), jnp.float32)]), jnp.int32)
    hh = CH // 2
    return jnp.concatenate([tgt.reshape(-1, 2, hh), src.reshape(-1, 2, hh),
                            vbits.reshape(-1, 2, hh)], axis=1)


def _combine_body(accl_ref, accu_ref, x_ref, wlt_ref, out_ref):
    accl = accl_ref[...]
    accu = accu_ref[...]
    sl = accl[:, D:D + 1]
    su = accu[:, D:D + 1]
    invl = jnp.where(sl > 0, 1.0 / sl, 0.0)
    invu = jnp.where(su > 0, 1.0 / su, 0.0)
    skip = jnp.dot(x_ref[...], wlt_ref[...], preferred_element_type=jnp.float32)
    out_ref[...] = jax.nn.relu(
        accl[:, :D] * invl + accu[:, :D] * invu + skip * (1.0 + 1e-6))


def _combine(acc_l, acc_u, x, W_lin_T):
    nb = N // BN
    return pl.pallas_call(
        _combine_body,
        grid=(nb,),
        in_specs=[
            pl.BlockSpec((BN, DE), lambda b: (b, 0)),
            pl.BlockSpec((BN, DE), lambda b: (b, 0)),
            pl.BlockSpec((BN, D), lambda b: (b, 0)),
            pl.BlockSpec((D, D), lambda b: (0, 0)),
        ],
        out_specs=pl.BlockSpec((BN, D), lambda b: (b, 0)),
        out_shape=jax.ShapeDtypeStruct((N, D), jnp.float32),
    )(acc_l, acc_u, x, W_lin_T)


def kernel(x, lower_indices, lower_values, upper_indices, upper_values,
           W_lower, att_lower, W_upper, att_upper, W_lin):
    W2 = jnp.stack([W_lower, W_upper])
    att2 = jnp.stack([att_lower.reshape(2, D).T, att_upper.reshape(2, D).T])
    xme_l, xme_u, a_l, a_u = _prep(x, W2, att2)
    a_l_t = jnp.transpose(a_l)
    a_u_t = jnp.transpose(a_u)
    zeros = jnp.zeros((NP, DE), jnp.float32)
    edata_l = _pack_edges(lower_indices, lower_values)
    edata_u = _pack_edges(upper_indices, upper_values)
    acc_l, acc_u = _sc_sparse(xme_l, xme_u, a_l_t, a_u_t,
                              edata_l, edata_u, zeros)
    return _combine(acc_l, acc_u, x, W_lin.T)
